# baseline (speedup 1.0000x reference)
import functools
import math

import numpy as np
import jax
import jax.numpy as jnp
from jax import lax
from jax.experimental import pallas as pl
from jax.experimental.pallas import tpu as pltpu

_S, _D, _E, _C = 8192, 768, 64, 128


def _threefry_uniform(seed: int, n: int) -> np.ndarray:
    k1 = np.uint32(np.uint64(seed) >> np.uint64(32))
    k2 = np.uint32(np.uint64(seed) & np.uint64(0xFFFFFFFF))
    i = np.arange(n, dtype=np.uint64)
    x1 = (i >> np.uint64(32)).astype(np.uint32)
    x2 = (i & np.uint64(0xFFFFFFFF)).astype(np.uint32)
    R0 = [13, 15, 26, 6]
    R1 = [17, 29, 16, 24]
    ks = [k1, k2, np.uint32(k1 ^ k2 ^ np.uint32(0x1BD11BDA))]
    x1 = (x1 + ks[0]).astype(np.uint32)
    x2 = (x2 + ks[1]).astype(np.uint32)

    def rotl(v, r):
        return ((v << np.uint32(r)) | (v >> np.uint32(32 - r))).astype(np.uint32)

    for r in range(5):
        for rot in (R0 if r % 2 == 0 else R1):
            x1 = (x1 + x2).astype(np.uint32)
            x2 = rotl(x2, rot)
            x2 = (x2 ^ x1).astype(np.uint32)
        x1 = (x1 + ks[(r + 1) % 3]).astype(np.uint32)
        x2 = (x2 + ks[(r + 2) % 3] + np.uint32(r + 1)).astype(np.uint32)
    bits = x1 ^ x2
    return (((bits >> np.uint32(9)) | np.uint32(0x3F800000)).view(np.float32)
            - np.float32(1.0))


def _rank_const() -> np.ndarray:
    noise = _threefry_uniform(1, _S * _E).reshape(_S, _E)
    order = np.argsort(-noise, axis=0, kind="stable")
    rank = np.empty((_S, _E), dtype=np.int32)
    np.put_along_axis(rank, order,
                      np.arange(_S, dtype=np.int32)[:, None], axis=0)
    return rank


_RANK = _rank_const()


def _route_kernel(x_ref, w_ref, r_ref,
                  idx_ref, loc_ref, gval_ref, sel_ref, cnt_ref, laux_ref):
    x = x_ref[...]
    w = w_ref[...]
    logits = lax.dot_general(x, w, (((1,), (1,)), ((), ())),
                             preferred_element_type=jnp.float32)
    m = jnp.max(logits, axis=1, keepdims=True)
    ex = jnp.exp(logits - m)
    se = jnp.sum(ex, axis=1, keepdims=True)
    gates_colsum = jnp.sum(ex / se, axis=0, keepdims=True)

    lane = lax.broadcasted_iota(jnp.int32, (_S, _E), 1)
    idx = jnp.min(jnp.where(logits == m, lane, _E), axis=1, keepdims=True)
    mask1 = lane == idx
    cnt = jnp.sum(mask1.astype(jnp.int32), axis=0, keepdims=True)

    r = r_ref[...]

    def bs_body(_, lohi):
        lo, hi = lohi
        mid = (lo + hi + 1) // 2
        c = jnp.sum((mask1 & (r < mid)).astype(jnp.int32),
                    axis=0, keepdims=True)
        good = c <= _C
        return jnp.where(good, mid, lo), jnp.where(good, hi, mid - 1)

    lo0 = jnp.zeros((1, _E), jnp.int32)
    hi0 = jnp.full((1, _E), _S, jnp.int32)
    lo, _ = lax.fori_loop(0, 14, bs_body, (lo0, hi0))
    sel2d = mask1 & (r < lo)

    selF = sel2d.astype(jnp.float32)
    ch = 512
    t0 = lax.broadcasted_iota(jnp.int32, (ch, ch), 0)
    t1 = lax.broadcasted_iota(jnp.int32, (ch, ch), 1)
    tri = (t0 >= t1).astype(jnp.float32)
    off = jnp.zeros((1, _E), jnp.float32)
    loc_parts = []
    for i in range(_S // ch):
        chunk = selF[i * ch:(i + 1) * ch]
        within = lax.dot_general(tri, chunk, (((1,), (0,)), ((), ())),
                                 preferred_element_type=jnp.float32)
        locs = within + off - 1.0
        loc_parts.append(jnp.sum(chunk * locs, axis=1, keepdims=True))
        off = off + jnp.sum(chunk, axis=0, keepdims=True)
    loc_tok = jnp.concatenate(loc_parts, axis=0).astype(jnp.int32)
    sel_tok = jnp.sum(sel2d.astype(jnp.int32), axis=1, keepdims=True)

    idx_ref[...] = idx
    loc_ref[...] = loc_tok
    gval_ref[...] = 1.0 / se
    sel_ref[...] = sel_tok
    cnt_ref[...] = cnt
    laux_ref[...] = (jnp.sum(gates_colsum * cnt.astype(jnp.float32))
                     * (_E / (_S * _S))).reshape(1, 1)


_BT = 256


def _hit_mask(idx_ref, loc_ref, sel_ref, bt):
    idxv = idx_ref[...]
    locv = loc_ref[...]
    selv = sel_ref[...]
    e_io = lax.broadcasted_iota(jnp.int32, (bt, _E, _C), 1)
    c_io = lax.broadcasted_iota(jnp.int32, (bt, _E, _C), 2)
    return ((e_io == idxv[:, :, None]) & (c_io == locv[:, :, None])
            & (selv[:, :, None] > 0))


def _comb_kernel(idx_ref, loc_ref, gval_ref, sel_ref, comb_ref):
    hit = _hit_mask(idx_ref, loc_ref, sel_ref, _BT)
    comb_ref[...] = jnp.where(hit, gval_ref[...][:, :, None], 0.0)


def _disp_kernel(idx_ref, loc_ref, sel_ref, w32_ref):
    idxv = idx_ref[...]
    locv = loc_ref[...]
    selv = sel_ref[...]
    e_io = lax.broadcasted_iota(jnp.int32, (_BT, _E, _C // 4), 1)
    w_io = lax.broadcasted_iota(jnp.int32, (_BT, _E, _C // 4), 2)
    word = jnp.int32(1) << (8 * (locv[:, :, None] & 3))
    hit = ((e_io == idxv[:, :, None]) & (w_io == (locv[:, :, None] >> 2))
           & (selv[:, :, None] > 0))
    w32_ref[...] = jnp.where(hit, word, 0)


def kernel(input, W):
    x = input.astype(jnp.float32)
    r = jnp.asarray(_RANK)

    meta_shapes = [
        jax.ShapeDtypeStruct((_S, 1), jnp.int32),
        jax.ShapeDtypeStruct((_S, 1), jnp.int32),
        jax.ShapeDtypeStruct((_S, 1), jnp.float32),
        jax.ShapeDtypeStruct((_S, 1), jnp.int32),
        jax.ShapeDtypeStruct((1, _E), jnp.int32),
        jax.ShapeDtypeStruct((1, 1), jnp.float32),
    ]
    idx, loc, gval, sel, cnt, laux = pl.pallas_call(
        _route_kernel,
        out_shape=meta_shapes,
    )(x, W, r)

    nb = _S // _BT
    col = pl.BlockSpec((_BT, 1), lambda i: (i, 0))
    out3d = pl.BlockSpec((_BT, _E, _C), lambda i: (i, 0, 0))
    comb = pl.pallas_call(
        _comb_kernel,
        grid=(nb,),
        in_specs=[col, col, col, col],
        out_specs=out3d,
        out_shape=jax.ShapeDtypeStruct((_S, _E, _C), jnp.float32),
    )(idx, loc, gval, sel)
    w32 = pl.pallas_call(
        _disp_kernel,
        grid=(nb,),
        in_specs=[col, col, col],
        out_specs=pl.BlockSpec((_BT, _E, _C // 4), lambda i: (i, 0, 0)),
        out_shape=jax.ShapeDtypeStruct((_S, _E, _C // 4), jnp.int32),
    )(idx, loc, sel)
    disp = lax.bitcast_convert_type(w32, jnp.int8).reshape(_S, _E, _C) != 0

    return laux[0, 0], comb, disp, cnt[0]


# speedup vs baseline: 3.1876x; 3.1876x over previous
import functools
import math

import numpy as np
import jax
import jax.numpy as jnp
from jax import lax
from jax.experimental import pallas as pl
from jax.experimental.pallas import tpu as pltpu

_S, _D, _E, _C = 8192, 768, 64, 128


def _threefry_uniform(seed: int, n: int) -> np.ndarray:
    k1 = np.uint32(np.uint64(seed) >> np.uint64(32))
    k2 = np.uint32(np.uint64(seed) & np.uint64(0xFFFFFFFF))
    i = np.arange(n, dtype=np.uint64)
    x1 = (i >> np.uint64(32)).astype(np.uint32)
    x2 = (i & np.uint64(0xFFFFFFFF)).astype(np.uint32)
    R0 = [13, 15, 26, 6]
    R1 = [17, 29, 16, 24]
    ks = [k1, k2, np.uint32(k1 ^ k2 ^ np.uint32(0x1BD11BDA))]
    x1 = (x1 + ks[0]).astype(np.uint32)
    x2 = (x2 + ks[1]).astype(np.uint32)

    def rotl(v, r):
        return ((v << np.uint32(r)) | (v >> np.uint32(32 - r))).astype(np.uint32)

    for r in range(5):
        for rot in (R0 if r % 2 == 0 else R1):
            x1 = (x1 + x2).astype(np.uint32)
            x2 = rotl(x2, rot)
            x2 = (x2 ^ x1).astype(np.uint32)
        x1 = (x1 + ks[(r + 1) % 3]).astype(np.uint32)
        x2 = (x2 + ks[(r + 2) % 3] + np.uint32(r + 1)).astype(np.uint32)
    bits = x1 ^ x2
    return (((bits >> np.uint32(9)) | np.uint32(0x3F800000)).view(np.float32)
            - np.float32(1.0))


def _rank_const() -> np.ndarray:
    noise = _threefry_uniform(1, _S * _E).reshape(_S, _E)
    order = np.argsort(-noise, axis=0, kind="stable")
    rank = np.empty((_S, _E), dtype=np.int32)
    np.put_along_axis(rank, order,
                      np.arange(_S, dtype=np.int32)[:, None], axis=0)
    return rank


_RANK = _rank_const()


def _route_kernel(x_ref, w_ref, r_ref,
                  idx_ref, loc_ref, gval_ref, sel_ref, cnt_ref, laux_ref):
    x = x_ref[...]
    w = w_ref[...]
    logits = lax.dot_general(x, w, (((1,), (1,)), ((), ())),
                             preferred_element_type=jnp.float32)
    m = jnp.max(logits, axis=1, keepdims=True)
    ex = jnp.exp(logits - m)
    se = jnp.sum(ex, axis=1, keepdims=True)
    gates_colsum = jnp.sum(ex / se, axis=0, keepdims=True)

    lane = lax.broadcasted_iota(jnp.int32, (_S, _E), 1)
    idx = jnp.min(jnp.where(logits == m, lane, _E), axis=1, keepdims=True)
    mask1 = lane == idx
    cnt = jnp.sum(mask1.astype(jnp.int32), axis=0, keepdims=True)

    r = r_ref[...]

    def bs_body(_, lohi):
        lo, hi = lohi
        mid = (lo + hi + 1) // 2
        c = jnp.sum((mask1 & (r < mid)).astype(jnp.int32),
                    axis=0, keepdims=True)
        good = c <= _C
        return jnp.where(good, mid, lo), jnp.where(good, hi, mid - 1)

    lo0 = jnp.zeros((1, _E), jnp.int32)
    hi0 = jnp.full((1, _E), _S, jnp.int32)
    lo, _ = lax.fori_loop(0, 14, bs_body, (lo0, hi0))
    sel2d = mask1 & (r < lo)

    selF = sel2d.astype(jnp.float32)
    ch = 512
    t0 = lax.broadcasted_iota(jnp.int32, (ch, ch), 0)
    t1 = lax.broadcasted_iota(jnp.int32, (ch, ch), 1)
    tri = (t0 >= t1).astype(jnp.float32)
    off = jnp.zeros((1, _E), jnp.float32)
    loc_parts = []
    for i in range(_S // ch):
        chunk = selF[i * ch:(i + 1) * ch]
        within = lax.dot_general(tri, chunk, (((1,), (0,)), ((), ())),
                                 preferred_element_type=jnp.float32)
        locs = within + off - 1.0
        loc_parts.append(jnp.sum(chunk * locs, axis=1, keepdims=True))
        off = off + jnp.sum(chunk, axis=0, keepdims=True)
    loc_tok = jnp.concatenate(loc_parts, axis=0).astype(jnp.int32)
    sel_tok = jnp.sum(sel2d.astype(jnp.int32), axis=1, keepdims=True)

    idx_ref[...] = idx
    loc_ref[...] = loc_tok
    gval_ref[...] = 1.0 / se
    sel_ref[...] = sel_tok
    cnt_ref[...] = cnt
    laux_ref[...] = (jnp.sum(gates_colsum * cnt.astype(jnp.float32))
                     * (_E / (_S * _S))).reshape(1, 1)


_BT = 256


def _hit_mask(idx_ref, loc_ref, sel_ref, bt):
    idxv = idx_ref[...]
    locv = loc_ref[...]
    selv = sel_ref[...]
    e_io = lax.broadcasted_iota(jnp.int32, (bt, _E, _C), 1)
    c_io = lax.broadcasted_iota(jnp.int32, (bt, _E, _C), 2)
    return ((e_io == idxv[:, :, None]) & (c_io == locv[:, :, None])
            & (selv[:, :, None] > 0))


def _comb_kernel(idx_ref, loc_ref, gval_ref, sel_ref, comb_ref):
    hit = _hit_mask(idx_ref, loc_ref, sel_ref, _BT)
    comb_ref[...] = jnp.where(hit, gval_ref[...][:, :, None], 0.0)


def kernel(input, W):
    x = input.astype(jnp.float32)
    r = jnp.asarray(_RANK)

    meta_shapes = [
        jax.ShapeDtypeStruct((_S, 1), jnp.int32),
        jax.ShapeDtypeStruct((_S, 1), jnp.int32),
        jax.ShapeDtypeStruct((_S, 1), jnp.float32),
        jax.ShapeDtypeStruct((_S, 1), jnp.int32),
        jax.ShapeDtypeStruct((1, _E), jnp.int32),
        jax.ShapeDtypeStruct((1, 1), jnp.float32),
    ]
    idx, loc, gval, sel, cnt, laux = pl.pallas_call(
        _route_kernel,
        out_shape=meta_shapes,
    )(x, W, r)

    nb = _S // _BT
    col = pl.BlockSpec((_BT, 1), lambda i: (i, 0))
    out3d = pl.BlockSpec((_BT, _E, _C), lambda i: (i, 0, 0))
    comb = pl.pallas_call(
        _comb_kernel,
        grid=(nb,),
        in_specs=[col, col, col, col],
        out_specs=out3d,
        out_shape=jax.ShapeDtypeStruct((_S, _E, _C), jnp.float32),
    )(idx, loc, gval, sel)
    disp = comb.astype(jnp.bool_)

    return laux[0, 0], comb, disp, cnt[0]


# speedup vs baseline: 3.1939x; 1.0020x over previous
import functools
import math

import numpy as np
import jax
import jax.numpy as jnp
from jax import lax
from jax.experimental import pallas as pl
from jax.experimental.pallas import tpu as pltpu

_S, _D, _E, _C = 8192, 768, 64, 128


def _threefry_uniform(seed: int, n: int) -> np.ndarray:
    k1 = np.uint32(np.uint64(seed) >> np.uint64(32))
    k2 = np.uint32(np.uint64(seed) & np.uint64(0xFFFFFFFF))
    i = np.arange(n, dtype=np.uint64)
    x1 = (i >> np.uint64(32)).astype(np.uint32)
    x2 = (i & np.uint64(0xFFFFFFFF)).astype(np.uint32)
    R0 = [13, 15, 26, 6]
    R1 = [17, 29, 16, 24]
    ks = [k1, k2, np.uint32(k1 ^ k2 ^ np.uint32(0x1BD11BDA))]
    x1 = (x1 + ks[0]).astype(np.uint32)
    x2 = (x2 + ks[1]).astype(np.uint32)

    def rotl(v, r):
        return ((v << np.uint32(r)) | (v >> np.uint32(32 - r))).astype(np.uint32)

    for r in range(5):
        for rot in (R0 if r % 2 == 0 else R1):
            x1 = (x1 + x2).astype(np.uint32)
            x2 = rotl(x2, rot)
            x2 = (x2 ^ x1).astype(np.uint32)
        x1 = (x1 + ks[(r + 1) % 3]).astype(np.uint32)
        x2 = (x2 + ks[(r + 2) % 3] + np.uint32(r + 1)).astype(np.uint32)
    bits = x1 ^ x2
    return (((bits >> np.uint32(9)) | np.uint32(0x3F800000)).view(np.float32)
            - np.float32(1.0))


def _rank_const() -> np.ndarray:
    noise = _threefry_uniform(1, _S * _E).reshape(_S, _E)
    order = np.argsort(-noise, axis=0, kind="stable")
    rank = np.empty((_S, _E), dtype=np.int32)
    np.put_along_axis(rank, order,
                      np.arange(_S, dtype=np.int32)[:, None], axis=0)
    return rank


_RANK = _rank_const()


def _route_kernel(x_ref, w_ref, r_ref,
                  idx_ref, loc_ref, gval_ref, sel_ref, cnt_ref, laux_ref):
    x = x_ref[...]
    w = w_ref[...]
    logits = lax.dot_general(x, w, (((1,), (1,)), ((), ())),
                             preferred_element_type=jnp.float32)
    m = jnp.max(logits, axis=1, keepdims=True)
    ex = jnp.exp(logits - m)
    se = jnp.sum(ex, axis=1, keepdims=True)
    gates_colsum = jnp.sum(ex / se, axis=0, keepdims=True)

    lane = lax.broadcasted_iota(jnp.int32, (_S, _E), 1)
    idx = jnp.min(jnp.where(logits == m, lane, _E), axis=1, keepdims=True)
    mask1 = lane == idx
    cnt = jnp.sum(mask1.astype(jnp.int32), axis=0, keepdims=True)

    r = r_ref[...]

    def bs_body(_, lohi):
        lo, hi = lohi
        mid = (lo + hi + 1) // 2
        c = jnp.sum((mask1 & (r < mid)).astype(jnp.int32),
                    axis=0, keepdims=True)
        good = c <= _C
        return jnp.where(good, mid, lo), jnp.where(good, hi, mid - 1)

    lo0 = jnp.zeros((1, _E), jnp.int32)
    hi0 = jnp.full((1, _E), _S, jnp.int32)
    lo, _ = lax.fori_loop(0, 14, bs_body, (lo0, hi0))
    sel2d = mask1 & (r < lo)

    selF = sel2d.astype(jnp.float32)
    ch = 512
    t0 = lax.broadcasted_iota(jnp.int32, (ch, ch), 0)
    t1 = lax.broadcasted_iota(jnp.int32, (ch, ch), 1)
    tri = (t0 >= t1).astype(jnp.float32)
    off = jnp.zeros((1, _E), jnp.float32)
    loc_parts = []
    for i in range(_S // ch):
        chunk = selF[i * ch:(i + 1) * ch]
        within = lax.dot_general(tri, chunk, (((1,), (0,)), ((), ())),
                                 preferred_element_type=jnp.float32)
        locs = within + off - 1.0
        loc_parts.append(jnp.sum(chunk * locs, axis=1, keepdims=True))
        off = off + jnp.sum(chunk, axis=0, keepdims=True)
    loc_tok = jnp.concatenate(loc_parts, axis=0).astype(jnp.int32)
    sel_tok = jnp.sum(sel2d.astype(jnp.int32), axis=1, keepdims=True)

    idx_ref[...] = idx
    loc_ref[...] = loc_tok
    gval_ref[...] = 1.0 / se
    sel_ref[...] = sel_tok
    cnt_ref[...] = cnt
    laux_ref[...] = (jnp.sum(gates_colsum * cnt.astype(jnp.float32))
                     * (_E / (_S * _S))).reshape(1, 1)


_BT = 512


def _hit_mask(idx_ref, loc_ref, sel_ref, bt):
    idxv = idx_ref[...]
    locv = loc_ref[...]
    selv = sel_ref[...]
    e_io = lax.broadcasted_iota(jnp.int32, (bt, _E, _C), 1)
    c_io = lax.broadcasted_iota(jnp.int32, (bt, _E, _C), 2)
    return ((e_io == idxv[:, :, None]) & (c_io == locv[:, :, None])
            & (selv[:, :, None] > 0))


def _comb_kernel(idx_ref, loc_ref, gval_ref, sel_ref, comb_ref):
    hit = _hit_mask(idx_ref, loc_ref, sel_ref, _BT)
    comb_ref[...] = jnp.where(hit, gval_ref[...][:, :, None], 0.0)


def kernel(input, W):
    x = input.astype(jnp.float32)
    r = jnp.asarray(_RANK)

    meta_shapes = [
        jax.ShapeDtypeStruct((_S, 1), jnp.int32),
        jax.ShapeDtypeStruct((_S, 1), jnp.int32),
        jax.ShapeDtypeStruct((_S, 1), jnp.float32),
        jax.ShapeDtypeStruct((_S, 1), jnp.int32),
        jax.ShapeDtypeStruct((1, _E), jnp.int32),
        jax.ShapeDtypeStruct((1, 1), jnp.float32),
    ]
    idx, loc, gval, sel, cnt, laux = pl.pallas_call(
        _route_kernel,
        out_shape=meta_shapes,
    )(x, W, r)

    nb = _S // _BT
    col = pl.BlockSpec((_BT, 1), lambda i: (i, 0))
    out3d = pl.BlockSpec((_BT, _E, _C), lambda i: (i, 0, 0))
    comb = pl.pallas_call(
        _comb_kernel,
        grid=(nb,),
        in_specs=[col, col, col, col],
        out_specs=out3d,
        out_shape=jax.ShapeDtypeStruct((_S, _E, _C), jnp.float32),
    )(idx, loc, gval, sel)
    disp = comb.astype(jnp.bool_)

    return laux[0, 0], comb, disp, cnt[0]
